# baseline (speedup 1.0000x reference)
import functools

import jax
import jax.numpy as jnp
from jax import lax
from jax.experimental import pallas as pl
from jax.experimental.pallas import tpu as pltpu
from jax.experimental.pallas import tpu_sc as plsc

D = 128
NP = 4096
PW = 50
NC = 2
NS = 16
NW = NC * NS
GP = 8
GCH = GP * PW
SCH = 80
NSTR = GCH // SCH
NSPLIT = 4

_mesh = plsc.VectorSubcoreMesh(core_axis_name="c", subcore_axis_name="s")


def _make_gather_kernel(np_part):
    ppw = np_part // NW
    bpw = ppw * PW
    ng = ppw // GP
    assert ng >= 2 and ng % 2 == 0 and ppw % GP == 0

    @functools.partial(
        pl.kernel,
        mesh=_mesh,
        out_type=jax.ShapeDtypeStruct((np_part, PW, D), jnp.float32),
        scratch_types=[
            pltpu.VMEM((bpw,), jnp.int32),
            pltpu.VMEM((GCH, D), jnp.float32),
            pltpu.VMEM((GCH, D), jnp.float32),
            pltpu.SemaphoreType.DMA,
            pltpu.SemaphoreType.DMA,
        ],
    )
    def _gather_kernel(x_hbm, embd_hbm, out_hbm, idx_v, buf_a, buf_b, gsem, ssem):
        wid = lax.axis_index("s") * NC + lax.axis_index("c")
        base = pl.multiple_of(wid * bpw, 8)
        pbase = wid * ppw
        pltpu.sync_copy(x_hbm.at[pl.ds(base, bpw)], idx_v)

        def gather_desc(g, buf, s):
            off = pl.multiple_of(g * GCH + s * SCH, 8)
            return pltpu.make_async_copy(
                embd_hbm.at[idx_v.at[pl.ds(off, SCH)]],
                buf.at[pl.ds(s * SCH, SCH)],
                gsem,
            )

        def store_desc(g, buf, p):
            return pltpu.make_async_copy(
                buf.at[pl.ds(p * PW, PW)],
                out_hbm.at[pbase + g * GP + p],
                ssem,
            )

        def fire_gathers(g, buf):
            for s in range(NSTR):
                gather_desc(g, buf, s).start()

        def wait_gathers(g, buf):
            for s in range(NSTR):
                gather_desc(g, buf, s).wait()

        def fire_stores(g, buf):
            for p in range(GP):
                store_desc(g, buf, p).start()

        def wait_stores(g, buf):
            for p in range(GP):
                store_desc(g, buf, p).wait()

        fire_gathers(0, buf_a)
        wait_gathers(0, buf_a)
        fire_gathers(1, buf_b)
        fire_stores(0, buf_a)

        def step(g, cur, other):
            wait_gathers(g, cur)
            wait_stores(g - 1, other)
            fire_gathers(g + 1, other)
            fire_stores(g, cur)

        def body(m, carry):
            step(2 * m + 1, buf_b, buf_a)
            step(2 * m + 2, buf_a, buf_b)
            return carry

        lax.fori_loop(0, (ng - 2) // 2, body, 0)

        wait_gathers(ng - 1, buf_b)
        wait_stores(ng - 2, buf_a)
        fire_stores(ng - 1, buf_b)
        wait_stores(ng - 1, buf_b)

    return _gather_kernel


_NP_PART = NP // NSPLIT
_part_kernel = _make_gather_kernel(_NP_PART)


def kernel(x, embd):
    flat = x.reshape(-1).astype(jnp.int32)
    bp = _NP_PART * PW
    parts = [
        _part_kernel(lax.slice(flat, (i * bp,), ((i + 1) * bp,)), embd)
        for i in range(NSPLIT)
    ]
    return jnp.concatenate(parts, axis=0)


# speedup vs baseline: 3.2158x; 3.2158x over previous
import functools

import jax
import jax.numpy as jnp
from jax import lax
from jax.experimental import pallas as pl
from jax.experimental.pallas import tpu as pltpu
from jax.experimental.pallas import tpu_sc as plsc

D = 128
NI = 4096
NJ = 50
B = NI * NJ
NC = 2
NS = 16
NW = NC * NS
BPW = B // NW
SCH = 128
GCH = 2 * SCH
NG = BPW // GCH

_mesh = plsc.VectorSubcoreMesh(core_axis_name="c", subcore_axis_name="s")


@functools.partial(
    pl.kernel,
    mesh=_mesh,
    out_type=jax.ShapeDtypeStruct((NJ, NI, D), jnp.float32),
    scratch_types=[
        pltpu.VMEM((BPW,), jnp.int32),
        pltpu.VMEM((GCH, D), jnp.float32),
        pltpu.VMEM((GCH, D), jnp.float32),
        pltpu.SemaphoreType.DMA,
        pltpu.SemaphoreType.DMA,
    ],
)
def _gather_kernel(xt_hbm, embd_hbm, out_hbm, idx_v, buf_a, buf_b, gsem, ssem):
    wid = lax.axis_index("s") * NC + lax.axis_index("c")
    base = pl.multiple_of(wid * BPW, GCH)
    pltpu.sync_copy(xt_hbm.at[pl.ds(base, BPW)], idx_v)

    def gather_desc(g, buf, s):
        off = pl.multiple_of(g * GCH + s * SCH, 8)
        return pltpu.make_async_copy(
            embd_hbm.at[idx_v.at[pl.ds(off, SCH)]],
            buf.at[pl.ds(s * SCH, SCH)],
            gsem,
        )

    def store_desc(g, buf):
        n0 = base + g * GCH
        j = n0 // NI
        i = pl.multiple_of(n0 % NI, GCH)
        return pltpu.make_async_copy(buf, out_hbm.at[j, pl.ds(i, GCH)], ssem)

    def fire_gathers(g, buf):
        for s in range(2):
            gather_desc(g, buf, s).start()

    def wait_gathers(g, buf):
        for s in range(2):
            gather_desc(g, buf, s).wait()

    fire_gathers(0, buf_a)
    wait_gathers(0, buf_a)
    fire_gathers(1, buf_b)
    store_desc(0, buf_a).start()

    def step(g, cur, other):
        wait_gathers(g, cur)
        store_desc(g - 1, other).wait()
        fire_gathers(g + 1, other)
        store_desc(g, cur).start()

    def body(m, carry):
        step(2 * m + 1, buf_b, buf_a)
        step(2 * m + 2, buf_a, buf_b)
        return carry

    lax.fori_loop(0, (NG - 3) // 2, body, 0)

    step(NG - 2, buf_b, buf_a)
    wait_gathers(NG - 1, buf_a)
    store_desc(NG - 2, buf_b).wait()
    store_desc(NG - 1, buf_a).start()
    store_desc(NG - 1, buf_a).wait()


def kernel(x, embd):
    flat_t = jnp.transpose(x).reshape(-1).astype(jnp.int32)
    out_t = _gather_kernel(flat_t, embd)
    return jnp.transpose(out_t, (1, 0, 2))


# speedup vs baseline: 3.3354x; 1.0372x over previous
import functools

import jax
import jax.numpy as jnp
from jax import lax
from jax.experimental import pallas as pl
from jax.experimental.pallas import tpu as pltpu
from jax.experimental.pallas import tpu_sc as plsc

D = 128
NI = 4096
NJ = 50
B = NI * NJ
NC = 2
NS = 16
NW = NC * NS
BPW = B // NW
SCH = 128
GCH = 2 * SCH
NG = BPW // GCH

_mesh = plsc.VectorSubcoreMesh(core_axis_name="c", subcore_axis_name="s")


@functools.partial(
    pl.kernel,
    mesh=_mesh,
    out_type=jax.ShapeDtypeStruct((NJ, NI, D), jnp.float32),
    scratch_types=[
        pltpu.VMEM((BPW,), jnp.int32),
        pltpu.VMEM((GCH, D), jnp.float32),
        pltpu.VMEM((GCH, D), jnp.float32),
        pltpu.VMEM((GCH, D), jnp.float32),
        pltpu.SemaphoreType.DMA,
        pltpu.SemaphoreType.DMA,
        pltpu.SemaphoreType.DMA,
        pltpu.SemaphoreType.DMA,
    ],
)
def _gather_kernel(
    xt_hbm, embd_hbm, out_hbm, idx_v, buf0, buf1, buf2, gsem0, gsem1, gsem2, ssem
):
    wid = lax.axis_index("s") * NC + lax.axis_index("c")
    base = pl.multiple_of(wid * BPW, GCH)
    pltpu.sync_copy(xt_hbm.at[pl.ds(base, BPW)], idx_v)

    bufs = (buf0, buf1, buf2)
    gsems = (gsem0, gsem1, gsem2)

    def gather_desc(g, k, s):
        off = pl.multiple_of(g * GCH + s * SCH, 8)
        return pltpu.make_async_copy(
            embd_hbm.at[idx_v.at[pl.ds(off, SCH)]],
            bufs[k].at[pl.ds(s * SCH, SCH)],
            gsems[k],
        )

    def store_desc(g, k):
        n0 = base + g * GCH
        j = n0 // NI
        i = pl.multiple_of(n0 % NI, GCH)
        return pltpu.make_async_copy(bufs[k], out_hbm.at[j, pl.ds(i, GCH)], ssem)

    def fire_gathers(g, k):
        for s in range(2):
            gather_desc(g, k, s).start()

    def wait_gathers(g, k):
        for s in range(2):
            gather_desc(g, k, s).wait()

    def step(g, k, first=False, fire_next=True):
        kn = (k + 2) % 3
        wait_gathers(g, k)
        if not first:
            store_desc(g - 1, kn).wait()
        if fire_next:
            fire_gathers(g + 2, kn)
        store_desc(g, k).start()

    fire_gathers(0, 0)
    fire_gathers(1, 1)
    step(0, 0, first=True)

    def body(m, carry):
        g = 3 * m + 1
        step(g, 1)
        step(g + 1, 2)
        step(g + 2, 0)
        return carry

    lax.fori_loop(0, (NG - 4) // 3, body, 0)

    step(NG - 3, 1)
    step(NG - 2, 2, fire_next=False)
    step(NG - 1, 0, fire_next=False)
    store_desc(NG - 1, 0).wait()


def kernel(x, embd):
    flat_t = jnp.transpose(x).reshape(-1).astype(jnp.int32)
    out_t = _gather_kernel(flat_t, embd)
    return jnp.transpose(out_t, (1, 0, 2))
